# baseline (speedup 1.0000x reference)
import functools

import jax
import jax.numpy as jnp
from jax.experimental import pallas as pl
from jax.experimental.pallas import tpu as pltpu

HID = 128
CUTOFF = 10.0
PI = 3.14159
TD = 64
TS = 64


def _layer_kernel(nt, slo_smem, shi_smem, p8_ref, h_ref, oh_ref, oht_ref,
                  ohdt_ref, ohdtf_ref, diagt_ref, Wa, Wb, GtC2, b1, W2t, b2U,
                  U1h, U1a, ub1, U2t, ub2, lng, lnb, w1c, db1c, out_ref):
    D = pl.program_id(0)
    d0 = D * TD
    sLo = slo_smem[D]
    sHi = shi_smem[D]

    hd = h_ref[pl.ds(d0, TD), :]
    npd8 = -p8_ref[pl.ds(d0, TD), :]
    Bhb1 = jnp.dot(hd, Wb[:, :], preferred_element_type=jnp.float32) \
        + b1[:, :]
    OH = oh_ref[:, :]
    OHT = oht_ref[:, :]
    OHDT = ohdt_ref[:, :]

    def tile_msgs(t, sel):
        s0 = jnp.minimum(t, nt - 1) * TS
        hs = h_ref[pl.ds(s0, TS), :]
        ps8 = p8_ref[pl.ds(s0, TS), :]
        Ah = jnp.dot(hs, Wa[:, :], preferred_element_type=jnp.float32)
        PSD = jnp.concatenate([ps8, npd8], axis=0)
        PDT = jax.lax.dot_general(
            PSD.astype(jnp.bfloat16), OHT, (((0,), (0,)), ((), ())),
            preferred_element_type=jnp.float32)
        d2 = (PDT[0:1, :] * PDT[0:1, :] + PDT[1:2, :] * PDT[1:2, :]
              + PDT[2:3, :] * PDT[2:3, :])
        dist = jnp.maximum(jnp.sqrt(d2), 1e-6)
        mT = ((PDT[3:4, :] == 0.0) & (dist < CUTOFF)).astype(jnp.float32)
        mT = mT * jnp.where(t == D, diagt_ref[:, :], sel)
        u = jnp.minimum(d2 * ((PI / CUTOFF) ** 2), PI * PI)
        c = (1.0, -0.5, 1.0 / 24, -1.0 / 720, 1.0 / 40320, -1.0 / 3628800,
             1.0 / 479001600, -1.0 / 87178291200, 1.0 / 20922789888000)
        pc = c[8]
        for k in (7, 6, 5, 4, 3, 2, 1, 0):
            pc = pc * u + c[k]
        cw = 0.5 * (1.0 + pc)
        df1 = dist * w1c[:, :] + db1c[:, :]
        df1 = df1 * jax.nn.sigmoid(df1)
        augT = jnp.concatenate([df1 * cw, cw, mT - 1.0], axis=0)
        aug = augT.astype(jnp.bfloat16).T
        AB = jnp.concatenate([Ah, Bhb1], axis=0)
        m1 = (jnp.dot(OH, AB.astype(jnp.bfloat16),
                      preferred_element_type=jnp.float32)
              + jnp.dot(aug, GtC2[:, :], preferred_element_type=jnp.float32)
              ).astype(jnp.bfloat16)
        m = m1 * jax.nn.sigmoid(m1)
        return m, mT

    def body(i, carry):
        red, cnt = carry
        ta = sLo + 2 * i
        tb = ta + 1
        ma, mTa = tile_msgs(ta, 1.0)
        mb, mTb = tile_msgs(tb, jnp.where(tb <= sHi, 1.0, 0.0))
        red = red + jnp.dot(OHDT, ma + mb,
                            preferred_element_type=jnp.float32)
        cnt = cnt + jax.lax.dot_general(
            ohdtf_ref[:, :], mTa + mTb, (((1,), (1,)), ((), ())),
            preferred_element_type=jnp.float32)
        return red, cnt

    red, cnt = jax.lax.fori_loop(
        0, (sHi - sLo) // 2 + 1, body,
        (jnp.zeros((TD, HID), jnp.float32), jnp.zeros((TD, 1), jnp.float32)))
    acc = jnp.dot(red, W2t[:, :], preferred_element_type=jnp.float32)

    u1 = (jnp.dot(hd, U1h[:, :], preferred_element_type=jnp.float32)
          + jnp.dot(acc, U1a[:, :], preferred_element_type=jnp.float32)
          + cnt * b2U[:, :]
          + ub1[:, :])
    u = u1 * jax.nn.sigmoid(u1)
    hn = jnp.dot(u, U2t[:, :], preferred_element_type=jnp.float32) + ub2[:, :]
    x = hd + hn
    mu = jnp.mean(x, axis=1, keepdims=True)
    xc = x - mu
    var = jnp.mean(xc * xc, axis=1, keepdims=True)
    out_ref[:, :] = xc * jax.lax.rsqrt(var + 1e-5) * lng[:, :] + lnb[:, :]


def _prep_layer(p):
    W1 = p["msg_W1"]
    A, B, C = W1[:, :HID], W1[:, HID:2 * HID], W1[:, 2 * HID:]
    GtC2 = jnp.concatenate(
        [(C @ p["de_W2"]).T,
         (C @ p["de_b2"]).reshape(1, HID),
         jnp.full((1, HID), 1e4, jnp.float32)],
        axis=0).astype(jnp.bfloat16)
    U1a = p["upd_W1"][:, HID:].T
    b2U = (p["msg_b2"].reshape(1, HID) @ U1a).reshape(1, HID)
    return (
        A.T, B.T,
        GtC2,
        p["msg_b1"].reshape(1, HID),
        p["msg_W2"].T,
        b2U,
        p["upd_W1"][:, :HID].T,
        U1a,
        p["upd_b1"].reshape(1, HID),
        p["upd_W2"].T,
        p["upd_b2"].reshape(1, HID),
        p["ln_g"].reshape(1, HID),
        p["ln_b"].reshape(1, HID),
        p["de_W1"][:, 0].reshape(16, 1),
        p["de_b1"].reshape(16, 1),
    )


def _layer_call(h, p8, slo, shi, oh, oht, ohdt, ohdtf, diagt, weights):
    n = h.shape[0]
    nd = n // TD
    nt = n // TS
    full = lambda shape: pl.BlockSpec(
        shape, lambda i, s1, s2: (0,) * len(shape))
    wspecs = [full(w.shape) for w in weights]
    grid_spec = pltpu.PrefetchScalarGridSpec(
        num_scalar_prefetch=2,
        grid=(nd,),
        in_specs=[full((n, 8)), full((n, HID)), full(oh.shape),
                  full(oht.shape), full(ohdt.shape), full(ohdtf.shape),
                  full(diagt.shape)]
        + wspecs,
        out_specs=pl.BlockSpec((TD, HID), lambda i, s1, s2: (i, 0)),
    )
    return pl.pallas_call(
        functools.partial(_layer_kernel, nt),
        grid_spec=grid_spec,
        out_shape=jax.ShapeDtypeStruct((n, HID), jnp.float32),
    )(slo, shi, p8, h, oh, oht, ohdt, ohdtf, diagt, *weights)


def _one_hots():
    es = jnp.eye(TS, dtype=jnp.float32)
    ed = jnp.eye(TD, dtype=jnp.float32)
    ohs = jnp.repeat(es, TD, axis=0)
    ohd = jnp.tile(ed, (TS, 1))
    oh = jnp.concatenate([ohs, ohd], axis=1)
    diagt = 1.0 - (ohs[:, :TD] * ohd).sum(axis=1)[None, :]
    bf = jnp.bfloat16
    return oh.astype(bf), oh.T.astype(bf), ohd.T.astype(bf), ohd.T, diagt


def kernel(h, pos, batch_idx, params):
    n = h.shape[0]
    b32 = batch_idx.astype(jnp.int32)
    p8 = jnp.concatenate(
        [pos.astype(jnp.float32),
         b32.astype(jnp.float32)[:, None],
         jnp.zeros((n, 4), jnp.float32)], axis=1)
    gmins = b32[0::TD]
    gmaxs = b32[TD - 1::TD]
    slo = (jnp.searchsorted(b32, gmins, side="left") // TS).astype(jnp.int32)
    shi = ((jnp.searchsorted(b32, gmaxs, side="right") - 1) // TS).astype(
        jnp.int32)
    oh, oht, ohdt, ohdtf, diagt = _one_hots()
    for p in params["layers"]:
        h = _layer_call(h, p8, slo, shi, oh, oht, ohdt, ohdtf, diagt,
                        _prep_layer(p))
    return h


# speedup vs baseline: 1.0667x; 1.0667x over previous
import functools

import jax
import jax.numpy as jnp
from jax.experimental import pallas as pl
from jax.experimental.pallas import tpu as pltpu

HID = 128
CUTOFF = 10.0
PI = 3.14159
TD = 64
TS = 64


def _layer_kernel(nt, b_smem, p8_ref, h_ref, oh_ref, oht_ref,
                  ohdt_ref, ohdtf_ref, diagt_ref, Wa, Wb, GtC2, b1, W2t, b2U,
                  U1h, U1a, ub1, U2t, ub2, lng, lnb, w1c, db1c, out_ref):
    D = pl.program_id(0)
    d0 = D * TD

    hd = h_ref[pl.ds(d0, TD), :]
    npd8 = -p8_ref[pl.ds(d0, TD), :]
    Bhb1 = jnp.dot(hd, Wb[:, :], preferred_element_type=jnp.float32) \
        + b1[:, :]

    gmin = b_smem[d0]
    gmax = b_smem[d0 + TD - 1]
    sLo = jax.lax.while_loop(
        lambda t: jnp.logical_and(t > 0, b_smem[t * TS - 1] >= gmin),
        lambda t: t - 1, D)
    sHi = jax.lax.while_loop(
        lambda t: jnp.logical_and(t < nt - 1, b_smem[(t + 1) * TS] <= gmax),
        lambda t: t + 1, D)

    def tile_msgs(t, sel):
        s0 = jnp.minimum(t, nt - 1) * TS
        hs = h_ref[pl.ds(s0, TS), :]
        ps8 = p8_ref[pl.ds(s0, TS), :]
        Ah = jnp.dot(hs, Wa[:, :], preferred_element_type=jnp.float32)
        PSD = jnp.concatenate([ps8, npd8], axis=0)
        PDT = jax.lax.dot_general(
            PSD.astype(jnp.bfloat16), oht_ref[:, :], (((0,), (0,)), ((), ())),
            preferred_element_type=jnp.float32)
        d2 = (PDT[0:1, :] * PDT[0:1, :] + PDT[1:2, :] * PDT[1:2, :]
              + PDT[2:3, :] * PDT[2:3, :])
        dist = jnp.maximum(jnp.sqrt(d2), 1e-6)
        mT = ((PDT[3:4, :] == 0.0) & (dist < CUTOFF)).astype(jnp.float32)
        mT = mT * jnp.where(t == D, diagt_ref[:, :], sel)
        u = jnp.minimum(d2 * ((PI / CUTOFF) ** 2), PI * PI)
        c = (1.0, -0.5, 1.0 / 24, -1.0 / 720, 1.0 / 40320, -1.0 / 3628800,
             1.0 / 479001600, -1.0 / 87178291200, 1.0 / 20922789888000)
        pc = c[8]
        for k in (7, 6, 5, 4, 3, 2, 1, 0):
            pc = pc * u + c[k]
        cw = 0.5 * (1.0 + pc)
        df1 = dist * w1c[:, :] + db1c[:, :]
        df1 = df1 * jax.nn.sigmoid(df1)
        augT = jnp.concatenate([df1 * cw, cw, mT - 1.0], axis=0)
        aug = augT.astype(jnp.bfloat16).T
        AB = jnp.concatenate([Ah, Bhb1], axis=0)
        m1 = (jnp.dot(oh_ref[:, :], AB.astype(jnp.bfloat16),
                      preferred_element_type=jnp.float32)
              + jnp.dot(aug, GtC2[:, :], preferred_element_type=jnp.float32)
              ).astype(jnp.bfloat16)
        m = m1 * jax.nn.sigmoid(m1)
        return m, mT

    def body(i, carry):
        red, cnt = carry
        ta = sLo + 2 * i
        tb = ta + 1
        ma, mTa = tile_msgs(ta, 1.0)
        mb, mTb = tile_msgs(tb, jnp.where(tb <= sHi, 1.0, 0.0))
        red = red + jnp.dot(ohdt_ref[:, :], ma + mb,
                            preferred_element_type=jnp.float32)
        cnt = cnt + jax.lax.dot_general(
            ohdtf_ref[:, :], mTa + mTb, (((1,), (1,)), ((), ())),
            preferred_element_type=jnp.float32)
        return red, cnt

    red, cnt = jax.lax.fori_loop(
        0, (sHi - sLo) // 2 + 1, body,
        (jnp.zeros((TD, HID), jnp.float32), jnp.zeros((TD, 1), jnp.float32)))
    acc = jnp.dot(red, W2t[:, :], preferred_element_type=jnp.float32)

    u1 = (jnp.dot(hd, U1h[:, :], preferred_element_type=jnp.float32)
          + jnp.dot(acc, U1a[:, :], preferred_element_type=jnp.float32)
          + cnt * b2U[:, :]
          + ub1[:, :])
    u = u1 * jax.nn.sigmoid(u1)
    hn = jnp.dot(u, U2t[:, :], preferred_element_type=jnp.float32) + ub2[:, :]
    x = hd + hn
    mu = jnp.mean(x, axis=1, keepdims=True)
    xc = x - mu
    var = jnp.mean(xc * xc, axis=1, keepdims=True)
    out_ref[:, :] = xc * jax.lax.rsqrt(var + 1e-5) * lng[:, :] + lnb[:, :]


def _prep_layer(p):
    W1 = p["msg_W1"]
    A, B, C = W1[:, :HID], W1[:, HID:2 * HID], W1[:, 2 * HID:]
    GtC2 = jnp.concatenate(
        [(C @ p["de_W2"]).T,
         (C @ p["de_b2"]).reshape(1, HID),
         jnp.full((1, HID), 1e4, jnp.float32)],
        axis=0).astype(jnp.bfloat16)
    U1a = p["upd_W1"][:, HID:].T
    b2U = (p["msg_b2"].reshape(1, HID) @ U1a).reshape(1, HID)
    return (
        A.T, B.T,
        GtC2,
        p["msg_b1"].reshape(1, HID),
        p["msg_W2"].T,
        b2U,
        p["upd_W1"][:, :HID].T,
        U1a,
        p["upd_b1"].reshape(1, HID),
        p["upd_W2"].T,
        p["upd_b2"].reshape(1, HID),
        p["ln_g"].reshape(1, HID),
        p["ln_b"].reshape(1, HID),
        p["de_W1"][:, 0].reshape(16, 1),
        p["de_b1"].reshape(16, 1),
    )


def _layer_call(h, p8, b32, oh, oht, ohdt, ohdtf, diagt, weights):
    n = h.shape[0]
    nd = n // TD
    nt = n // TS
    full = lambda shape: pl.BlockSpec(shape, lambda i, b: (0,) * len(shape))
    wspecs = [full(w.shape) for w in weights]
    grid_spec = pltpu.PrefetchScalarGridSpec(
        num_scalar_prefetch=1,
        grid=(nd,),
        in_specs=[full((n, 8)), full((n, HID)), full(oh.shape),
                  full(oht.shape), full(ohdt.shape), full(ohdtf.shape),
                  full(diagt.shape)]
        + wspecs,
        out_specs=pl.BlockSpec((TD, HID), lambda i, b: (i, 0)),
    )
    return pl.pallas_call(
        functools.partial(_layer_kernel, nt),
        grid_spec=grid_spec,
        out_shape=jax.ShapeDtypeStruct((n, HID), jnp.float32),
    )(b32, p8, h, oh, oht, ohdt, ohdtf, diagt, *weights)


def _one_hots():
    es = jnp.eye(TS, dtype=jnp.float32)
    ed = jnp.eye(TD, dtype=jnp.float32)
    ohs = jnp.repeat(es, TD, axis=0)
    ohd = jnp.tile(ed, (TS, 1))
    oh = jnp.concatenate([ohs, ohd], axis=1)
    diagt = 1.0 - (ohs[:, :TD] * ohd).sum(axis=1)[None, :]
    bf = jnp.bfloat16
    return oh.astype(bf), oh.T.astype(bf), ohd.T.astype(bf), ohd.T, diagt


def kernel(h, pos, batch_idx, params):
    n = h.shape[0]
    b32 = batch_idx.astype(jnp.int32)
    p8 = jnp.concatenate(
        [pos.astype(jnp.float32),
         b32.astype(jnp.float32)[:, None],
         jnp.zeros((n, 4), jnp.float32)], axis=1)
    oh, oht, ohdt, ohdtf, diagt = _one_hots()
    for p in params["layers"]:
        h = _layer_call(h, p8, b32, oh, oht, ohdt, ohdtf, diagt,
                        _prep_layer(p))
    return h


# speedup vs baseline: 1.0842x; 1.0165x over previous
import functools

import jax
import jax.numpy as jnp
from jax.experimental import pallas as pl
from jax.experimental.pallas import tpu as pltpu

HID = 128
CUTOFF = 10.0
PI = 3.14159
TD = 64
TS = 64


def _layer_kernel(nt, b_smem, p8_ref, h_ref, oh_ref, oht_ref,
                  ohdt_ref, ohdtf_ref, diagt_ref, Wa, Wb, GtC2, b1, W2t, b2U,
                  U1h, U1a, ub1, U2t, ub2, lng, lnb, w1c, db1c, out_ref):
    D = pl.program_id(0)
    d0 = D * TD

    hd = h_ref[pl.ds(d0, TD), :]
    npd8 = -p8_ref[pl.ds(d0, TD), :]
    Bhb1 = jnp.dot(hd, Wb[:, :], preferred_element_type=jnp.float32) \
        + b1[:, :]

    gmin = b_smem[d0]
    gmax = b_smem[d0 + TD - 1]
    sLo = jax.lax.while_loop(
        lambda t: jnp.logical_and(t > 0, b_smem[t * TS - 1] >= gmin),
        lambda t: t - 1, D)
    sHi = jax.lax.while_loop(
        lambda t: jnp.logical_and(t < nt - 1, b_smem[(t + 1) * TS] <= gmax),
        lambda t: t + 1, D)

    def tile_msgs(t, sel):
        s0 = jnp.minimum(t, nt - 1) * TS
        hs = h_ref[pl.ds(s0, TS), :]
        ps8 = p8_ref[pl.ds(s0, TS), :]
        Ah = jnp.dot(hs, Wa[:, :], preferred_element_type=jnp.float32)
        PSD = jnp.concatenate([ps8, npd8], axis=0)
        PDT = jax.lax.dot_general(
            PSD.astype(jnp.bfloat16), oht_ref[:, :], (((0,), (0,)), ((), ())),
            preferred_element_type=jnp.float32)
        d2 = (PDT[0:1, :] * PDT[0:1, :] + PDT[1:2, :] * PDT[1:2, :]
              + PDT[2:3, :] * PDT[2:3, :])
        dist = jnp.maximum(jnp.sqrt(d2), 1e-6)
        mT = ((PDT[3:4, :] == 0.0) & (dist < CUTOFF)).astype(jnp.float32)
        mT = mT * jnp.where(t == D, diagt_ref[:, :], sel)
        u = jnp.minimum(d2 * ((PI / CUTOFF) ** 2), PI * PI)
        c = (1.0, -0.5, 1.0 / 24, -1.0 / 720, 1.0 / 40320, -1.0 / 3628800,
             1.0 / 479001600, -1.0 / 87178291200, 1.0 / 20922789888000)
        pc = c[8]
        for k in (7, 6, 5, 4, 3, 2, 1, 0):
            pc = pc * u + c[k]
        cw = 0.5 * (1.0 + pc)
        df1 = dist * w1c[:, :] + db1c[:, :]
        df1 = df1 * jax.nn.sigmoid(df1)
        augT = jnp.concatenate([df1 * cw, cw, mT - 1.0], axis=0)
        aug = augT.astype(jnp.bfloat16).T
        AB = jnp.concatenate([Ah, Bhb1], axis=0)
        lhs = jnp.concatenate([oh_ref[:, :], aug], axis=1)
        rhs = jnp.concatenate([AB.astype(jnp.bfloat16), GtC2[:, :]], axis=0)
        m1 = jnp.dot(lhs, rhs,
                     preferred_element_type=jnp.float32).astype(jnp.bfloat16)
        m = m1 * jax.nn.sigmoid(m1)
        return m, mT

    def body(i, carry):
        red, cnt = carry
        ta = sLo + 2 * i
        tb = ta + 1
        ma, mTa = tile_msgs(ta, 1.0)
        mb, mTb = tile_msgs(tb, jnp.where(tb <= sHi, 1.0, 0.0))
        red = red + jnp.dot(ohdt_ref[:, :], ma + mb,
                            preferred_element_type=jnp.float32)
        cnt = cnt + jax.lax.dot_general(
            ohdtf_ref[:, :], mTa + mTb, (((1,), (1,)), ((), ())),
            preferred_element_type=jnp.float32)
        return red, cnt

    red, cnt = jax.lax.fori_loop(
        0, (sHi - sLo) // 2 + 1, body,
        (jnp.zeros((TD, HID), jnp.float32), jnp.zeros((TD, 1), jnp.float32)))
    acc = jnp.dot(red, W2t[:, :], preferred_element_type=jnp.float32)

    u1 = (jnp.dot(hd, U1h[:, :], preferred_element_type=jnp.float32)
          + jnp.dot(acc, U1a[:, :], preferred_element_type=jnp.float32)
          + cnt * b2U[:, :]
          + ub1[:, :])
    u = u1 * jax.nn.sigmoid(u1)
    hn = jnp.dot(u, U2t[:, :], preferred_element_type=jnp.float32) + ub2[:, :]
    x = hd + hn
    mu = jnp.mean(x, axis=1, keepdims=True)
    xc = x - mu
    var = jnp.mean(xc * xc, axis=1, keepdims=True)
    out_ref[:, :] = xc * jax.lax.rsqrt(var + 1e-5) * lng[:, :] + lnb[:, :]


def _prep_layer(p):
    W1 = p["msg_W1"]
    A, B, C = W1[:, :HID], W1[:, HID:2 * HID], W1[:, 2 * HID:]
    GtC2 = jnp.concatenate(
        [(C @ p["de_W2"]).T,
         (C @ p["de_b2"]).reshape(1, HID),
         jnp.full((1, HID), 1e4, jnp.float32)],
        axis=0).astype(jnp.bfloat16)
    U1a = p["upd_W1"][:, HID:].T
    b2U = (p["msg_b2"].reshape(1, HID) @ U1a).reshape(1, HID)
    return (
        A.T, B.T,
        GtC2,
        p["msg_b1"].reshape(1, HID),
        p["msg_W2"].T,
        b2U,
        p["upd_W1"][:, :HID].T,
        U1a,
        p["upd_b1"].reshape(1, HID),
        p["upd_W2"].T,
        p["upd_b2"].reshape(1, HID),
        p["ln_g"].reshape(1, HID),
        p["ln_b"].reshape(1, HID),
        p["de_W1"][:, 0].reshape(16, 1),
        p["de_b1"].reshape(16, 1),
    )


def _layer_call(h, p8, b32, oh, oht, ohdt, ohdtf, diagt, weights):
    n = h.shape[0]
    nd = n // TD
    nt = n // TS
    full = lambda shape: pl.BlockSpec(shape, lambda i, b: (0,) * len(shape))
    wspecs = [full(w.shape) for w in weights]
    grid_spec = pltpu.PrefetchScalarGridSpec(
        num_scalar_prefetch=1,
        grid=(nd,),
        in_specs=[full((n, 8)), full((n, HID)), full(oh.shape),
                  full(oht.shape), full(ohdt.shape), full(ohdtf.shape),
                  full(diagt.shape)]
        + wspecs,
        out_specs=pl.BlockSpec((TD, HID), lambda i, b: (i, 0)),
    )
    return pl.pallas_call(
        functools.partial(_layer_kernel, nt),
        grid_spec=grid_spec,
        out_shape=jax.ShapeDtypeStruct((n, HID), jnp.float32),
    )(b32, p8, h, oh, oht, ohdt, ohdtf, diagt, *weights)


def _one_hots():
    es = jnp.eye(TS, dtype=jnp.float32)
    ed = jnp.eye(TD, dtype=jnp.float32)
    ohs = jnp.repeat(es, TD, axis=0)
    ohd = jnp.tile(ed, (TS, 1))
    oh = jnp.concatenate([ohs, ohd], axis=1)
    diagt = 1.0 - (ohs[:, :TD] * ohd).sum(axis=1)[None, :]
    bf = jnp.bfloat16
    return oh.astype(bf), oh.T.astype(bf), ohd.T.astype(bf), ohd.T, diagt


def kernel(h, pos, batch_idx, params):
    n = h.shape[0]
    b32 = batch_idx.astype(jnp.int32)
    p8 = jnp.concatenate(
        [pos.astype(jnp.float32),
         b32.astype(jnp.float32)[:, None],
         jnp.zeros((n, 4), jnp.float32)], axis=1)
    oh, oht, ohdt, ohdtf, diagt = _one_hots()
    for p in params["layers"]:
        h = _layer_call(h, p8, b32, oh, oht, ohdt, ohdtf, diagt,
                        _prep_layer(p))
    return h


# speedup vs baseline: 1.0969x; 1.0117x over previous
import functools

import jax
import jax.numpy as jnp
from jax.experimental import pallas as pl
from jax.experimental.pallas import tpu as pltpu

HID = 128
CUTOFF = 10.0
PI = 3.14159
TD = 64
TS = 64


def _gnn_kernel(nt, b_smem, p8_ref, h_ref, oh_ref, oht_ref,
                ohdt_ref, ohdtf_ref, diagt_ref, wa_ref, wb_ref, gt_ref,
                b1_ref, w2_ref, b2u_ref, u1h_ref, u1a_ref, ub1_ref, u2_ref,
                ub2_ref, lng_ref, lnb_ref, w1_ref, db1_ref, out_ref, scr_ref):
    L = pl.program_id(0)
    D = pl.program_id(1)
    d0 = D * TD
    r0 = L * HID

    def hrows(start, rows):
        return jnp.where(L == 0, h_ref[pl.ds(start, rows), :],
                         scr_ref[pl.ds(start, rows), :])

    hd = hrows(d0, TD)
    npd8 = -p8_ref[pl.ds(d0, TD), :]
    Bhb1 = jnp.dot(hd, wb_ref[pl.ds(r0, HID), :],
                   preferred_element_type=jnp.float32) \
        + b1_ref[pl.ds(L * 8, 1), :]
    GtC2 = gt_ref[pl.ds(L * 24, 18), :]
    Wa = wa_ref[pl.ds(r0, HID), :]
    w1c = w1_ref[pl.ds(L * 16, 16), :]
    db1c = db1_ref[pl.ds(L * 16, 16), :]

    gmin = b_smem[d0]
    gmax = b_smem[d0 + TD - 1]
    sLo = jax.lax.while_loop(
        lambda t: jnp.logical_and(t > 0, b_smem[t * TS - 1] >= gmin),
        lambda t: t - 1, D)
    sHi = jax.lax.while_loop(
        lambda t: jnp.logical_and(t < nt - 1, b_smem[(t + 1) * TS] <= gmax),
        lambda t: t + 1, D)

    def tile_msgs(t, sel):
        s0 = jnp.minimum(t, nt - 1) * TS
        hs = hrows(s0, TS)
        ps8 = p8_ref[pl.ds(s0, TS), :]
        Ah = jnp.dot(hs, Wa, preferred_element_type=jnp.float32)
        PSD = jnp.concatenate([ps8, npd8], axis=0)
        PDT = jax.lax.dot_general(
            PSD.astype(jnp.bfloat16), oht_ref[:, :], (((0,), (0,)), ((), ())),
            preferred_element_type=jnp.float32)
        d2 = (PDT[0:1, :] * PDT[0:1, :] + PDT[1:2, :] * PDT[1:2, :]
              + PDT[2:3, :] * PDT[2:3, :])
        dist = jnp.maximum(jnp.sqrt(d2), 1e-6)
        mT = ((PDT[3:4, :] == 0.0) & (dist < CUTOFF)).astype(jnp.float32)
        mT = mT * jnp.where(t == D, diagt_ref[:, :], sel)
        u = jnp.minimum(d2 * ((PI / CUTOFF) ** 2), PI * PI)
        c = (1.0, -0.5, 1.0 / 24, -1.0 / 720, 1.0 / 40320, -1.0 / 3628800,
             1.0 / 479001600, -1.0 / 87178291200, 1.0 / 20922789888000)
        pc = c[8]
        for k in (7, 6, 5, 4, 3, 2, 1, 0):
            pc = pc * u + c[k]
        cw = 0.5 * (1.0 + pc)
        df1 = dist * w1c + db1c
        df1 = df1 * jax.nn.sigmoid(df1)
        augT = jnp.concatenate([df1 * cw, cw, mT - 1.0], axis=0)
        aug = augT.astype(jnp.bfloat16).T
        AB = jnp.concatenate([Ah, Bhb1], axis=0)
        lhs = jnp.concatenate([oh_ref[:, :], aug], axis=1)
        rhs = jnp.concatenate([AB.astype(jnp.bfloat16), GtC2], axis=0)
        m1 = jnp.dot(lhs, rhs,
                     preferred_element_type=jnp.float32).astype(jnp.bfloat16)
        m = m1 * jax.nn.sigmoid(m1)
        return m, mT

    def body(i, carry):
        red, cnt = carry
        ta = sLo + 2 * i
        tb = ta + 1
        ma, mTa = tile_msgs(ta, 1.0)
        mb, mTb = tile_msgs(tb, jnp.where(tb <= sHi, 1.0, 0.0))
        red = red + jnp.dot(ohdt_ref[:, :], ma + mb,
                            preferred_element_type=jnp.float32)
        cnt = cnt + jax.lax.dot_general(
            ohdtf_ref[:, :], mTa + mTb, (((1,), (1,)), ((), ())),
            preferred_element_type=jnp.float32)
        return red, cnt

    red, cnt = jax.lax.fori_loop(
        0, (sHi - sLo) // 2 + 1, body,
        (jnp.zeros((TD, HID), jnp.float32), jnp.zeros((TD, 1), jnp.float32)))
    acc = jnp.dot(red, w2_ref[pl.ds(r0, HID), :],
                  preferred_element_type=jnp.float32)

    u1 = (jnp.dot(hd, u1h_ref[pl.ds(r0, HID), :],
                  preferred_element_type=jnp.float32)
          + jnp.dot(acc, u1a_ref[pl.ds(r0, HID), :],
                    preferred_element_type=jnp.float32)
          + cnt * b2u_ref[pl.ds(L * 8, 1), :]
          + ub1_ref[pl.ds(L * 8, 1), :])
    u = u1 * jax.nn.sigmoid(u1)
    hn = jnp.dot(u, u2_ref[pl.ds(r0, HID), :],
                 preferred_element_type=jnp.float32) \
        + ub2_ref[pl.ds(L * 8, 1), :]
    x = hd + hn
    mu = jnp.mean(x, axis=1, keepdims=True)
    xc = x - mu
    var = jnp.mean(xc * xc, axis=1, keepdims=True)
    out = xc * jax.lax.rsqrt(var + 1e-5) * lng_ref[pl.ds(L * 8, 1), :] \
        + lnb_ref[pl.ds(L * 8, 1), :]

    @pl.when(L == 0)
    def _():
        scr_ref[pl.ds(d0, TD), :] = out

    @pl.when(L == 1)
    def _():
        out_ref[:, :] = out


def _row8(x):
    return jnp.pad(x.reshape(1, HID), ((0, 7), (0, 0)))


def _prep_layer(p):
    W1 = p["msg_W1"]
    A, B, C = W1[:, :HID], W1[:, HID:2 * HID], W1[:, 2 * HID:]
    GtC2 = jnp.pad(jnp.concatenate(
        [(C @ p["de_W2"]).T,
         (C @ p["de_b2"]).reshape(1, HID),
         jnp.full((1, HID), 1e4, jnp.float32)],
        axis=0), ((0, 6), (0, 0))).astype(jnp.bfloat16)
    U1a = p["upd_W1"][:, HID:].T
    b2U = p["msg_b2"].reshape(1, HID) @ U1a
    return (
        A.T, B.T,
        GtC2,
        _row8(p["msg_b1"]),
        p["msg_W2"].T,
        _row8(b2U),
        p["upd_W1"][:, :HID].T,
        U1a,
        _row8(p["upd_b1"]),
        p["upd_W2"].T,
        _row8(p["upd_b2"]),
        _row8(p["ln_g"]),
        _row8(p["ln_b"]),
        p["de_W1"][:, 0].reshape(16, 1),
        p["de_b1"].reshape(16, 1),
    )


def _one_hots():
    es = jnp.eye(TS, dtype=jnp.float32)
    ed = jnp.eye(TD, dtype=jnp.float32)
    ohs = jnp.repeat(es, TD, axis=0)
    ohd = jnp.tile(ed, (TS, 1))
    oh = jnp.concatenate([ohs, ohd], axis=1)
    diagt = 1.0 - (ohs[:, :TD] * ohd).sum(axis=1)[None, :]
    bf = jnp.bfloat16
    return oh.astype(bf), oh.T.astype(bf), ohd.T.astype(bf), ohd.T, diagt


def kernel(h, pos, batch_idx, params):
    n = h.shape[0]
    nd = n // TD
    nt = n // TS
    b32 = batch_idx.astype(jnp.int32)
    p8 = jnp.concatenate(
        [pos.astype(jnp.float32),
         b32.astype(jnp.float32)[:, None],
         jnp.zeros((n, 4), jnp.float32)], axis=1)
    oh, oht, ohdt, ohdtf, diagt = _one_hots()
    pls = [_prep_layer(p) for p in params["layers"]]
    weights = [jnp.concatenate(ws, axis=0) for ws in zip(*pls)]

    full = lambda shape: pl.BlockSpec(shape, lambda l, i, b: (0,) * len(shape))
    wspecs = [full(w.shape) for w in weights]
    grid_spec = pltpu.PrefetchScalarGridSpec(
        num_scalar_prefetch=1,
        grid=(len(pls), nd),
        in_specs=[full((n, 8)), full((n, HID)), full(oh.shape),
                  full(oht.shape), full(ohdt.shape), full(ohdtf.shape),
                  full(diagt.shape)]
        + wspecs,
        out_specs=pl.BlockSpec((TD, HID), lambda l, i, b: (i, 0)),
        scratch_shapes=[pltpu.VMEM((n, HID), jnp.float32)],
    )
    return pl.pallas_call(
        functools.partial(_gnn_kernel, nt),
        grid_spec=grid_spec,
        out_shape=jax.ShapeDtypeStruct((n, HID), jnp.float32),
    )(b32, p8, h, oh, oht, ohdt, ohdtf, diagt, *weights)
